# baseline (speedup 1.0000x reference)
import functools

import jax
import jax.numpy as jnp
from jax import lax
from jax.experimental import pallas as pl
from jax.experimental.pallas import tpu as pltpu
from jax.experimental.pallas import tpu_sc as plsc

MOM = 0.5
C = 100
K = 8
CK = C * K
D = 256
BANK = 100000
B = 4096
MB = 512
NBLK = B // MB

NC = 2
NS = 16
NW = NC * NS
BPW = B // NW

NS1 = 16
RPW = 6256
RPW_LAST = BANK - (NS1 - 1) * RPW
SPW = B // NS1

@functools.cache
def _sc_kernels():
    mesh2 = plsc.VectorSubcoreMesh(core_axis_name="c", subcore_axis_name="s",
                                   num_cores=NC, num_subcores=NS)
    mesh1 = plsc.VectorSubcoreMesh(core_axis_name="c", subcore_axis_name="s",
                                   num_cores=1, num_subcores=NS)

    @functools.partial(
        pl.kernel,
        out_type=jax.ShapeDtypeStruct((B, D), jnp.float32),
        mesh=mesh2,
        scratch_types=[
            pltpu.VMEM((BPW,), jnp.int32),
            pltpu.VMEM((BPW, D), jnp.float32),
            pltpu.SemaphoreType.DMA,
        ],
    )
    def sc_gather(bank_hbm, ids_hbm, out_hbm, idx_v, rows_v, sem):
        wid = lax.axis_index("s") * NC + lax.axis_index("c")
        base = wid * BPW
        pltpu.sync_copy(ids_hbm.at[pl.ds(base, BPW)], idx_v)
        pltpu.async_copy(bank_hbm.at[idx_v], rows_v, sem).wait()
        pltpu.sync_copy(rows_v, out_hbm.at[pl.ds(base, BPW)])

    @functools.partial(
        pl.kernel,
        out_type=jax.ShapeDtypeStruct((BANK, D), jnp.float32),
        mesh=mesh1,
        scratch_types=[
            pltpu.VMEM((BPW,), jnp.int32),
            pltpu.VMEM((BPW,), jnp.int32),
            pltpu.VMEM((SPW, D), jnp.float32),
            pltpu.SemaphoreType.DMA,
        ],
    )
    def sc_bank(bank_hbm, ids_hbm, fnew_hbm, out_hbm, idx_a, idx_b, rows_v,
                sem):
        sid = lax.axis_index("s")
        r0 = sid * RPW

        @pl.when(sid < NS1 - 1)
        def _():
            pltpu.sync_copy(bank_hbm.at[pl.ds(r0, RPW)],
                            out_hbm.at[pl.ds(r0, RPW)])

        @pl.when(sid == NS1 - 1)
        def _():
            pltpu.sync_copy(bank_hbm.at[pl.ds(r0, RPW_LAST)],
                            out_hbm.at[pl.ds(r0, RPW_LAST)])

        plsc.subcore_barrier()
        pltpu.sync_copy(ids_hbm.at[pl.ds(sid * SPW, BPW)], idx_a)
        pltpu.sync_copy(ids_hbm.at[pl.ds(sid * SPW + BPW, BPW)], idx_b)
        pltpu.sync_copy(fnew_hbm.at[pl.ds(sid * SPW, SPW)], rows_v)
        cp0 = pltpu.async_copy(rows_v.at[pl.ds(0, BPW)],
                               out_hbm.at[idx_a], sem)
        cp1 = pltpu.async_copy(rows_v.at[pl.ds(BPW, BPW)],
                               out_hbm.at[idx_b], sem)
        cp0.wait()
        cp1.wait()

    @functools.partial(
        pl.kernel,
        out_type=jax.ShapeDtypeStruct((BANK,), jnp.int32),
        mesh=mesh1,
        scratch_types=[
            pltpu.VMEM((BANK,), jnp.int32),
            pltpu.VMEM((B,), jnp.int32),
            pltpu.VMEM((B,), jnp.int32),
        ],
        compiler_params=pltpu.CompilerParams(needs_layout_passes=False),
    )
    def sc_assigns(assigns_hbm, ids_hbm, na_hbm, out_hbm, asg_v, ids_v, na_v):
        sid = lax.axis_index("s")

        @pl.when(sid == 0)
        def _():
            pltpu.sync_copy(assigns_hbm, asg_v)
            pltpu.sync_copy(ids_hbm, ids_v)
            pltpu.sync_copy(na_hbm, na_v)

            def body(k, carry):
                idx = ids_v[pl.ds(k * 16, 16)]
                v = na_v[pl.ds(k * 16, 16)]
                plsc.store_scatter(asg_v, [idx], v)
                return carry

            lax.fori_loop(0, B // 16, body, 0)
            pltpu.sync_copy(asg_v, out_hbm)

    return sc_gather, sc_bank, sc_assigns


def _dense_body(gath_ref, feat_ref, out_ref, cm_ref, tgt_ref,
                fnew_ref, sim_ref, na_ref, cor_ref, los_ref):
    tgt = tgt_ref[0, 0, :]
    f = feat_ref[...]
    fn1 = f / jnp.maximum(jnp.sum(jnp.abs(f), axis=1, keepdims=True), 1e-12)
    fnew = (1.0 - MOM) * gath_ref[...] + MOM * fn1
    fnew = fnew / jnp.maximum(jnp.sum(jnp.abs(fnew), axis=1, keepdims=True),
                              1e-12)
    fnew_ref[...] = fnew

    sim = lax.dot_general(cm_ref[...], fnew, (((1,), (1,)), ((), ())),
                          preferred_element_type=jnp.float32)
    row_class = lax.broadcasted_iota(jnp.int32, (CK, MB), 0) // K
    mask = (row_class != tgt[None, :]).astype(jnp.float32)
    sim = sim - 10000.0 * mask
    sim_ref[...] = sim

    best = jnp.max(sim, axis=0, keepdims=True)
    ridx = lax.broadcasted_iota(jnp.int32, (CK, MB), 0)
    na = jnp.min(jnp.where(sim == best, ridx, CK), axis=0)
    na_ref[0, 0, :] = na

    o = out_ref[...]
    obest = jnp.max(o, axis=1, keepdims=True)
    cidx = lax.broadcasted_iota(jnp.int32, (MB, C), 1)
    oam = jnp.min(jnp.where(o == obest, cidx, C), axis=1)
    cor_ref[0, 0, :] = (oam == tgt).astype(jnp.int32)

    m = jnp.max(o, axis=1, keepdims=True)
    lse = jnp.log(jnp.sum(jnp.exp(o - m), axis=1)) + m[:, 0]
    pick = jnp.sum(jnp.where(cidx == tgt[:, None], o, 0.0), axis=1)
    los_ref[0, 0, :] = lse - pick


def _tc_dense(gath, feature, out, cm2, tgt3, interpret=False):
    return pl.pallas_call(
        _dense_body,
        grid=(NBLK,),
        in_specs=[
            pl.BlockSpec((MB, D), lambda i: (i, 0)),
            pl.BlockSpec((MB, D), lambda i: (i, 0)),
            pl.BlockSpec((MB, C), lambda i: (i, 0)),
            pl.BlockSpec((CK, D), lambda i: (0, 0)),
            pl.BlockSpec((1, 1, MB), lambda i: (i, 0, 0)),
        ],
        out_specs=[
            pl.BlockSpec((MB, D), lambda i: (i, 0)),
            pl.BlockSpec((CK, MB), lambda i: (0, i)),
            pl.BlockSpec((1, 1, MB), lambda i: (i, 0, 0)),
            pl.BlockSpec((1, 1, MB), lambda i: (i, 0, 0)),
            pl.BlockSpec((1, 1, MB), lambda i: (i, 0, 0)),
        ],
        out_shape=[
            jax.ShapeDtypeStruct((B, D), jnp.float32),
            jax.ShapeDtypeStruct((CK, B), jnp.float32),
            jax.ShapeDtypeStruct((NBLK, 1, MB), jnp.int32),
            jax.ShapeDtypeStruct((NBLK, 1, MB), jnp.int32),
            jax.ShapeDtypeStruct((NBLK, 1, MB), jnp.float32),
        ],
        interpret=interpret,
    )(gath, feature, out, cm2, tgt3)


def kernel(feature, out, feature_bank, cluster_means, target, ids, assigns):
    cm2 = cluster_means.reshape(CK, D)
    tgt3 = target.astype(jnp.int32).reshape(NBLK, 1, MB)
    ids = ids.astype(jnp.int32)

    sc_gather, sc_bank, sc_assigns = _sc_kernels()
    gath = sc_gather(feature_bank, ids)
    fnew, sim, na3, cor3, los3 = _tc_dense(gath, feature, out, cm2, tgt3)
    na = na3.reshape(B)
    bank_new = sc_bank(feature_bank, ids, fnew)
    asg_new = sc_assigns(assigns, ids, na)
    return (sim, bank_new, asg_new, na, cor3.reshape(B), los3.reshape(B))


# speedup vs baseline: 25.5873x; 25.5873x over previous
import functools

import jax
import jax.numpy as jnp
from jax import lax
from jax.experimental import pallas as pl
from jax.experimental.pallas import tpu as pltpu
from jax.experimental.pallas import tpu_sc as plsc

MOM = 0.5
C = 100
K = 8
CK = C * K
D = 256
BANK = 100000
B = 4096
MB = 512
NBLK = B // MB

NC = 2
NS = 16
NW = NC * NS
BPW = B // NW

NS1 = 16
RPW = 6256
RPW_LAST = BANK - (NS1 - 1) * RPW
SPW = B // NS1

@functools.cache
def _sc_kernels():
    mesh2 = plsc.VectorSubcoreMesh(core_axis_name="c", subcore_axis_name="s",
                                   num_cores=NC, num_subcores=NS)
    mesh1 = plsc.VectorSubcoreMesh(core_axis_name="c", subcore_axis_name="s",
                                   num_cores=1, num_subcores=NS)

    @functools.partial(
        pl.kernel,
        out_type=jax.ShapeDtypeStruct((B, D), jnp.float32),
        mesh=mesh2,
        scratch_types=[
            pltpu.VMEM((BPW,), jnp.int32),
            pltpu.VMEM((BPW, D), jnp.float32),
            pltpu.SemaphoreType.DMA,
        ],
    )
    def sc_gather(bank_hbm, ids_hbm, out_hbm, idx_v, rows_v, sem):
        wid = lax.axis_index("s") * NC + lax.axis_index("c")
        base = wid * BPW
        pltpu.sync_copy(ids_hbm.at[pl.ds(base, BPW)], idx_v)
        pltpu.async_copy(bank_hbm.at[idx_v], rows_v, sem).wait()
        pltpu.sync_copy(rows_v, out_hbm.at[pl.ds(base, BPW)])

    @functools.partial(
        pl.kernel,
        out_type=(),
        mesh=mesh2,
        scratch_types=[
            pltpu.VMEM((BPW,), jnp.int32),
            pltpu.VMEM((BPW, D), jnp.float32),
            pltpu.SemaphoreType.DMA,
        ],
    )
    def sc_bank(ids_hbm, fnew_hbm, bank_ref, idx_v, rows_v, sem):
        wid = lax.axis_index("s") * NC + lax.axis_index("c")
        base = wid * BPW
        pltpu.sync_copy(ids_hbm.at[pl.ds(base, BPW)], idx_v)
        pltpu.sync_copy(fnew_hbm.at[pl.ds(base, BPW)], rows_v)
        pltpu.async_copy(rows_v, bank_ref.at[idx_v], sem).wait()

    @functools.partial(
        pl.kernel,
        out_type=jax.ShapeDtypeStruct((BANK,), jnp.int32),
        mesh=mesh1,
        scratch_types=[
            pltpu.VMEM((BANK,), jnp.int32),
            pltpu.VMEM((B,), jnp.int32),
            pltpu.VMEM((B,), jnp.int32),
        ],
        compiler_params=pltpu.CompilerParams(needs_layout_passes=False),
    )
    def sc_assigns(assigns_hbm, ids_hbm, na_hbm, out_hbm, asg_v, ids_v, na_v):
        sid = lax.axis_index("s")

        @pl.when(sid == 0)
        def _():
            pltpu.sync_copy(assigns_hbm, asg_v)
            pltpu.sync_copy(ids_hbm, ids_v)
            pltpu.sync_copy(na_hbm, na_v)

            def body(k, carry):
                idx = ids_v[pl.ds(k * 16, 16)]
                v = na_v[pl.ds(k * 16, 16)]
                plsc.store_scatter(asg_v, [idx], v)
                return carry

            lax.fori_loop(0, B // 16, body, 0)
            pltpu.sync_copy(asg_v, out_hbm)

    return sc_gather, sc_bank, sc_assigns


def _dense_body(gath_ref, feat_ref, out_ref, cm_ref, tgt_ref,
                fnew_ref, sim_ref, na_ref, cor_ref, los_ref):
    tgt = tgt_ref[0, 0, :]
    f = feat_ref[...]
    fn1 = f / jnp.maximum(jnp.sum(jnp.abs(f), axis=1, keepdims=True), 1e-12)
    fnew = (1.0 - MOM) * gath_ref[...] + MOM * fn1
    fnew = fnew / jnp.maximum(jnp.sum(jnp.abs(fnew), axis=1, keepdims=True),
                              1e-12)
    fnew_ref[...] = fnew

    sim = lax.dot_general(cm_ref[...], fnew, (((1,), (1,)), ((), ())),
                          preferred_element_type=jnp.float32)
    row_class = lax.broadcasted_iota(jnp.int32, (CK, MB), 0) // K
    mask = (row_class != tgt[None, :]).astype(jnp.float32)
    sim = sim - 10000.0 * mask
    sim_ref[...] = sim

    best = jnp.max(sim, axis=0, keepdims=True)
    ridx = lax.broadcasted_iota(jnp.int32, (CK, MB), 0)
    na = jnp.min(jnp.where(sim == best, ridx, CK), axis=0)
    na_ref[0, 0, :] = na

    o = out_ref[...]
    obest = jnp.max(o, axis=1, keepdims=True)
    cidx = lax.broadcasted_iota(jnp.int32, (MB, C), 1)
    oam = jnp.min(jnp.where(o == obest, cidx, C), axis=1)
    cor_ref[0, 0, :] = (oam == tgt).astype(jnp.int32)

    m = jnp.max(o, axis=1, keepdims=True)
    lse = jnp.log(jnp.sum(jnp.exp(o - m), axis=1)) + m[:, 0]
    pick = jnp.sum(jnp.where(cidx == tgt[:, None], o, 0.0), axis=1)
    los_ref[0, 0, :] = lse - pick


def _tc_dense(gath, feature, out, cm2, tgt3, interpret=False):
    return pl.pallas_call(
        _dense_body,
        grid=(NBLK,),
        in_specs=[
            pl.BlockSpec((MB, D), lambda i: (i, 0)),
            pl.BlockSpec((MB, D), lambda i: (i, 0)),
            pl.BlockSpec((MB, C), lambda i: (i, 0)),
            pl.BlockSpec((CK, D), lambda i: (0, 0)),
            pl.BlockSpec((1, 1, MB), lambda i: (i, 0, 0)),
        ],
        out_specs=[
            pl.BlockSpec((MB, D), lambda i: (i, 0)),
            pl.BlockSpec((CK, MB), lambda i: (0, i)),
            pl.BlockSpec((1, 1, MB), lambda i: (i, 0, 0)),
            pl.BlockSpec((1, 1, MB), lambda i: (i, 0, 0)),
            pl.BlockSpec((1, 1, MB), lambda i: (i, 0, 0)),
        ],
        out_shape=[
            jax.ShapeDtypeStruct((B, D), jnp.float32),
            jax.ShapeDtypeStruct((CK, B), jnp.float32),
            jax.ShapeDtypeStruct((NBLK, 1, MB), jnp.int32),
            jax.ShapeDtypeStruct((NBLK, 1, MB), jnp.int32),
            jax.ShapeDtypeStruct((NBLK, 1, MB), jnp.float32),
        ],
        interpret=interpret,
    )(gath, feature, out, cm2, tgt3)


def kernel(feature, out, feature_bank, cluster_means, target, ids, assigns):
    cm2 = cluster_means.reshape(CK, D)
    tgt3 = target.astype(jnp.int32).reshape(NBLK, 1, MB)
    ids = ids.astype(jnp.int32)

    sc_gather, sc_bank, sc_assigns = _sc_kernels()
    gath = sc_gather(feature_bank, ids)
    fnew, sim, na3, cor3, los3 = _tc_dense(gath, feature, out, cm2, tgt3)
    na = na3.reshape(B)
    bank_ref = jax.new_ref(feature_bank)
    sc_bank(ids, fnew, bank_ref)
    bank_new = bank_ref[...]
    asg_new = sc_assigns(assigns, ids, na)
    return (sim, bank_new, asg_new, na, cor3.reshape(B), los3.reshape(B))
